# Initial kernel scaffold; baseline (speedup 1.0000x reference)
import jax
import jax.numpy as jnp
from jax.experimental import pallas as pl


def kernel(query, support, features, neighbors, kernel_points, weights, bias, gamma, beta):
    raise NotImplementedError("write your pallas kernel here")


# speedup vs baseline: 3.0387x; 3.0387x over previous
import functools

import jax
import jax.numpy as jnp
from jax.experimental import pallas as pl

B = 2
N = 8192
M = 8192
K = 32
P = 15
P_PAD = 16
IN_C = 64
OUT_C = 64
SIGMA = 1.0
EPS = 1e-5

Q = B * N
BLK = 128
NBLK = Q // BLK


def _stage1_kernel(q_ref, snb_ref, fnb_ref, kp_ref, w2_ref, bias_ref,
                   conv_ref, sums_ref):
    i = pl.program_id(0)

    dx = snb_ref[0] - q_ref[0][:, None]
    dy = snb_ref[1] - q_ref[1][:, None]
    dz = snb_ref[2] - q_ref[2][:, None]

    fnb = fnb_ref[...]

    infl_list = []
    for p in range(P):
        ddx = dx - kp_ref[p, 0]
        ddy = dy - kp_ref[p, 1]
        ddz = dz - kp_ref[p, 2]
        sq = ddx * ddx + ddy * ddy + ddz * ddz
        d = jnp.sqrt(sq)
        infl_list.append(jnp.maximum(1.0 - d / SIGMA, 0.0))
    infl_list.append(jnp.zeros_like(infl_list[0]))
    infl = jnp.stack(infl_list, axis=-1)

    acc = jnp.zeros((BLK, P_PAD, IN_C), dtype=jnp.float32)
    for k in range(K):
        acc = acc + infl[:, k, :, None] * fnb[:, k, None, :]
    weighted = acc.reshape(BLK, P_PAD * IN_C)

    conv = jnp.dot(weighted, w2_ref[...],
                   preferred_element_type=jnp.float32)

    nsum = jnp.sum(fnb, axis=-1)
    ncount = jnp.maximum(
        jnp.sum((jnp.abs(nsum) > 0.0).astype(jnp.float32), axis=-1), 1.0)
    conv = conv / ncount[:, None] + bias_ref[0][None, :]

    conv_ref[...] = conv

    rows = jnp.stack([jnp.sum(conv, axis=0),
                      jnp.sum(conv * conv, axis=0)], axis=0)
    rows = jnp.concatenate(
        [rows, jnp.zeros((2, 128 - OUT_C), jnp.float32)], axis=1)
    s = jnp.concatenate([rows, jnp.zeros((6, 128), jnp.float32)], axis=0)

    @pl.when(i == 0)
    def _():
        sums_ref[...] = s

    @pl.when(i != 0)
    def _():
        sums_ref[...] += s


def _stage2_kernel(conv_ref, sums_ref, gamma_ref, beta_ref, out_ref):
    mean = sums_ref[0, :OUT_C] / float(Q)
    var = sums_ref[1, :OUT_C] / float(Q) - mean * mean
    inv = jax.lax.rsqrt(var + EPS)
    xn = (conv_ref[...] - mean[None, :]) * inv[None, :]
    xn = xn * gamma_ref[0][None, :] + beta_ref[0][None, :]
    out_ref[...] = jnp.maximum(xn, 0.0)


@functools.partial(jax.jit, static_argnames=())
def kernel(query, support, features, neighbors, kernel_points, weights,
           bias, gamma, beta):
    nb_glob = (neighbors + jnp.arange(B, dtype=jnp.int32)[:, None, None] * M
               ).reshape(Q * K)
    f_flat = features.reshape(B * M, IN_C)
    s_flat = support.reshape(B * M, 3)

    fnb = jnp.take(f_flat, nb_glob, axis=0).reshape(Q, K, IN_C)
    snb = jnp.take(s_flat, nb_glob, axis=0).reshape(Q, K, 3)
    snb3 = jnp.transpose(snb, (2, 0, 1))
    q3 = jnp.transpose(query.reshape(Q, 3), (1, 0))

    kp_pad = jnp.zeros((P_PAD, 128), dtype=jnp.float32)
    kp_pad = kp_pad.at[:P, :3].set(kernel_points)

    w2 = jnp.concatenate(
        [weights.reshape(P * IN_C, OUT_C),
         jnp.zeros((IN_C, OUT_C), dtype=jnp.float32)], axis=0)

    bias2 = jnp.zeros((8, OUT_C), dtype=jnp.float32).at[0].set(bias)
    gamma2 = jnp.zeros((8, OUT_C), dtype=jnp.float32).at[0].set(gamma)
    beta2 = jnp.zeros((8, OUT_C), dtype=jnp.float32).at[0].set(beta)

    conv, sums = pl.pallas_call(
        _stage1_kernel,
        grid=(NBLK,),
        in_specs=[
            pl.BlockSpec((3, BLK), lambda i: (0, i)),
            pl.BlockSpec((3, BLK, K), lambda i: (0, i, 0)),
            pl.BlockSpec((BLK, K, IN_C), lambda i: (i, 0, 0)),
            pl.BlockSpec((P_PAD, 128), lambda i: (0, 0)),
            pl.BlockSpec((P_PAD * IN_C, OUT_C), lambda i: (0, 0)),
            pl.BlockSpec((8, OUT_C), lambda i: (0, 0)),
        ],
        out_specs=[
            pl.BlockSpec((BLK, OUT_C), lambda i: (i, 0)),
            pl.BlockSpec((8, 128), lambda i: (0, 0)),
        ],
        out_shape=[
            jax.ShapeDtypeStruct((Q, OUT_C), jnp.float32),
            jax.ShapeDtypeStruct((8, 128), jnp.float32),
        ],
    )(q3, snb3, fnb, kp_pad, w2, bias2)

    out = pl.pallas_call(
        _stage2_kernel,
        grid=(NBLK,),
        in_specs=[
            pl.BlockSpec((BLK, OUT_C), lambda i: (i, 0)),
            pl.BlockSpec((8, 128), lambda i: (0, 0)),
            pl.BlockSpec((8, OUT_C), lambda i: (0, 0)),
            pl.BlockSpec((8, OUT_C), lambda i: (0, 0)),
        ],
        out_specs=pl.BlockSpec((BLK, OUT_C), lambda i: (i, 0)),
        out_shape=jax.ShapeDtypeStruct((Q, OUT_C), jnp.float32),
    )(conv, sums, gamma2, beta2)

    return out.reshape(B, N, OUT_C)


# speedup vs baseline: 3.1097x; 1.0234x over previous
import functools

import jax
import jax.numpy as jnp
from jax import lax
from jax.experimental import pallas as pl
from jax.experimental.pallas import tpu as pltpu
from jax.experimental.pallas import tpu_sc as plsc

B = 2
N = 8192
M = 8192
K = 32
P = 15
P_PAD = 16
IN_C = 64
OUT_C = 64
SIGMA = 1.0
EPS = 1e-5

Q = B * N
ROWS = Q * K
NW = 32
RPW = ROWS // NW
CH = 128
NCH = RPW // CH

BLK = 64
NBLK = Q // BLK


def _sc_gather_body(f_hbm, s_hbm, nb_hbm, fnb_hbm, snb_hbm,
                    idx_v, frows, srows, semf, sems):
    wid = lax.axis_index("s") * 2 + lax.axis_index("c")
    base0 = wid * RPW

    def body(j, carry):
        base = base0 + j * CH
        pltpu.sync_copy(nb_hbm.at[pl.ds(base, CH)], idx_v)
        cf = pltpu.async_copy(f_hbm.at[idx_v], frows, semf)
        cs = pltpu.async_copy(s_hbm.at[idx_v], srows, sems)
        cf.wait()
        cs.wait()
        pltpu.sync_copy(frows, fnb_hbm.at[pl.ds(base, CH)])
        pltpu.sync_copy(srows, snb_hbm.at[pl.ds(base, CH)])
        return carry

    lax.fori_loop(0, NCH, body, 0)


def _make_sc_gather():
    return pl.kernel(
        _sc_gather_body,
        out_type=[jax.ShapeDtypeStruct((ROWS, IN_C), jnp.float32),
                  jax.ShapeDtypeStruct((ROWS, 16), jnp.float32)],
        mesh=plsc.VectorSubcoreMesh(core_axis_name="c",
                                    subcore_axis_name="s"),
        scratch_types=[pltpu.VMEM((CH,), jnp.int32),
                       pltpu.VMEM((CH, IN_C), jnp.float32),
                       pltpu.VMEM((CH, 16), jnp.float32),
                       pltpu.SemaphoreType.DMA,
                       pltpu.SemaphoreType.DMA],
        compiler_params=pltpu.CompilerParams(use_tc_tiling_on_sc=False),
    )


def _stage1_kernel(q_ref, snb_ref, fnb_ref, kp_ref, w2_ref, bias_ref,
                   conv_ref, sums_ref):
    i = pl.program_id(0)

    dx = snb_ref[:, :, 0] - q_ref[:, 0][:, None]
    dy = snb_ref[:, :, 1] - q_ref[:, 1][:, None]
    dz = snb_ref[:, :, 2] - q_ref[:, 2][:, None]

    fnb = fnb_ref[...]

    infl_list = []
    for p in range(P):
        ddx = dx - kp_ref[p, 0]
        ddy = dy - kp_ref[p, 1]
        ddz = dz - kp_ref[p, 2]
        sq = ddx * ddx + ddy * ddy + ddz * ddz
        d = jnp.sqrt(sq)
        infl_list.append(jnp.maximum(1.0 - d / SIGMA, 0.0))
    infl_list.append(jnp.zeros_like(infl_list[0]))
    infl = jnp.stack(infl_list, axis=-1)

    acc = jnp.zeros((BLK, P_PAD, IN_C), dtype=jnp.float32)
    for k in range(K):
        acc = acc + infl[:, k, :, None] * fnb[:, k, None, :]
    weighted = acc.reshape(BLK, P_PAD * IN_C)

    conv = jnp.dot(weighted, w2_ref[...],
                   preferred_element_type=jnp.float32)

    nsum = jnp.sum(fnb, axis=-1)
    ncount = jnp.maximum(
        jnp.sum((jnp.abs(nsum) > 0.0).astype(jnp.float32), axis=-1), 1.0)
    conv = conv / ncount[:, None] + bias_ref[0][None, :]

    conv_ref[...] = conv

    rows = jnp.stack([jnp.sum(conv, axis=0),
                      jnp.sum(conv * conv, axis=0)], axis=0)
    rows = jnp.concatenate(
        [rows, jnp.zeros((2, 128 - OUT_C), jnp.float32)], axis=1)
    s = jnp.concatenate([rows, jnp.zeros((6, 128), jnp.float32)], axis=0)

    @pl.when(i == 0)
    def _():
        sums_ref[...] = s

    @pl.when(i != 0)
    def _():
        sums_ref[...] += s


def _stage2_kernel(conv_ref, sums_ref, gamma_ref, beta_ref, out_ref):
    mean = sums_ref[0, :OUT_C] / float(Q)
    var = sums_ref[1, :OUT_C] / float(Q) - mean * mean
    inv = lax.rsqrt(var + EPS)
    xn = (conv_ref[...] - mean[None, :]) * inv[None, :]
    xn = xn * gamma_ref[0][None, :] + beta_ref[0][None, :]
    out_ref[...] = jnp.maximum(xn, 0.0)


def kernel(query, support, features, neighbors, kernel_points, weights,
           bias, gamma, beta):
    nb1d = (neighbors.astype(jnp.int32)
            + jnp.arange(B, dtype=jnp.int32)[:, None, None] * M
            ).reshape(ROWS)
    f_flat = features.reshape(B * M, IN_C)
    s_pad = jnp.concatenate(
        [support.reshape(B * M, 3),
         jnp.zeros((B * M, 13), jnp.float32)], axis=1)

    fnb, snb = _make_sc_gather()(f_flat, s_pad, nb1d)
    fnb = fnb.reshape(Q, K, IN_C)
    snb = snb.reshape(Q, K, 16)

    q8 = jnp.concatenate(
        [query.reshape(Q, 3), jnp.zeros((Q, 5), jnp.float32)], axis=1)

    kp_pad = jnp.zeros((P_PAD, 128), dtype=jnp.float32)
    kp_pad = kp_pad.at[:P, :3].set(kernel_points)

    w2 = jnp.concatenate(
        [weights.reshape(P * IN_C, OUT_C),
         jnp.zeros((IN_C, OUT_C), dtype=jnp.float32)], axis=0)

    bias2 = jnp.zeros((8, OUT_C), dtype=jnp.float32).at[0].set(bias)
    gamma2 = jnp.zeros((8, OUT_C), dtype=jnp.float32).at[0].set(gamma)
    beta2 = jnp.zeros((8, OUT_C), dtype=jnp.float32).at[0].set(beta)

    conv, sums = pl.pallas_call(
        _stage1_kernel,
        grid=(NBLK,),
        in_specs=[
            pl.BlockSpec((BLK, 8), lambda i: (i, 0)),
            pl.BlockSpec((BLK, K, 16), lambda i: (i, 0, 0)),
            pl.BlockSpec((BLK, K, IN_C), lambda i: (i, 0, 0)),
            pl.BlockSpec((P_PAD, 128), lambda i: (0, 0)),
            pl.BlockSpec((P_PAD * IN_C, OUT_C), lambda i: (0, 0)),
            pl.BlockSpec((8, OUT_C), lambda i: (0, 0)),
        ],
        out_specs=[
            pl.BlockSpec((BLK, OUT_C), lambda i: (i, 0)),
            pl.BlockSpec((8, 128), lambda i: (0, 0)),
        ],
        out_shape=[
            jax.ShapeDtypeStruct((Q, OUT_C), jnp.float32),
            jax.ShapeDtypeStruct((8, 128), jnp.float32),
        ],
    )(q8, snb, fnb, kp_pad, w2, bias2)

    out = pl.pallas_call(
        _stage2_kernel,
        grid=(NBLK,),
        in_specs=[
            pl.BlockSpec((BLK, OUT_C), lambda i: (i, 0)),
            pl.BlockSpec((8, 128), lambda i: (0, 0)),
            pl.BlockSpec((8, OUT_C), lambda i: (0, 0)),
            pl.BlockSpec((8, OUT_C), lambda i: (0, 0)),
        ],
        out_specs=pl.BlockSpec((BLK, OUT_C), lambda i: (i, 0)),
        out_shape=jax.ShapeDtypeStruct((Q, OUT_C), jnp.float32),
    )(conv, sums, gamma2, beta2)

    return out.reshape(B, N, OUT_C)


# speedup vs baseline: 9.0552x; 2.9119x over previous
import functools

import jax
import jax.numpy as jnp
from jax import lax
from jax.experimental import pallas as pl
from jax.experimental.pallas import tpu as pltpu
from jax.experimental.pallas import tpu_sc as plsc

B = 2
N = 8192
M = 8192
K = 32
P = 15
P_PAD = 16
IN_C = 64
OUT_C = 64
SIGMA = 1.0
EPS = 1e-5

Q = B * N
ROWS = Q * K
NW = 32
RPW = ROWS // NW
CH = 128
NCH = RPW // CH

BLK = 128
NBLK = Q // BLK


def _sc_gather_body(f_hbm, s_hbm, nb_hbm, fnb_hbm, snb_hbm,
                    idx_v, frows, srows, semf, sems):
    wid = lax.axis_index("s") * 2 + lax.axis_index("c")
    base0 = wid * RPW

    def body(j, carry):
        base = base0 + j * CH
        pltpu.sync_copy(nb_hbm.at[pl.ds(base, CH)], idx_v)
        cf = pltpu.async_copy(f_hbm.at[idx_v], frows, semf)
        cs = pltpu.async_copy(s_hbm.at[idx_v], srows, sems)
        cf.wait()
        cs.wait()
        pltpu.sync_copy(frows, fnb_hbm.at[pl.ds(base, CH)])
        pltpu.sync_copy(srows, snb_hbm.at[pl.ds(base, CH)])
        return carry

    lax.fori_loop(0, NCH, body, 0)


def _make_sc_gather():
    return pl.kernel(
        _sc_gather_body,
        out_type=[jax.ShapeDtypeStruct((ROWS, IN_C), jnp.float32),
                  jax.ShapeDtypeStruct((ROWS, 16), jnp.float32)],
        mesh=plsc.VectorSubcoreMesh(core_axis_name="c",
                                    subcore_axis_name="s"),
        scratch_types=[pltpu.VMEM((CH,), jnp.int32),
                       pltpu.VMEM((CH, IN_C), jnp.float32),
                       pltpu.VMEM((CH, 16), jnp.float32),
                       pltpu.SemaphoreType.DMA,
                       pltpu.SemaphoreType.DMA],
        compiler_params=pltpu.CompilerParams(use_tc_tiling_on_sc=False),
    )


def _stage1_kernel(q_ref, snb_ref, fnb_ref, kpt_ref, kpn_ref, w2_ref,
                   bias_ref, conv_ref, sums_ref):
    i = pl.program_id(0)

    qrep = jnp.broadcast_to(q_ref[...][:, None, :],
                            (BLK, K, 16)).reshape(BLK * K, 16)
    d3 = snb_ref[...] - qrep

    cross = jnp.dot(d3, kpt_ref[...],
                    preferred_element_type=jnp.float32)
    nrm = jnp.sum(d3 * d3, axis=1, keepdims=True)
    sq = jnp.maximum(nrm + kpn_ref[0][None, :] - 2.0 * cross, 0.0)
    d = jnp.sqrt(sq)
    infl = jnp.maximum(1.0 - d / SIGMA, 0.0)
    pmask = (lax.broadcasted_iota(jnp.int32, (1, 16), 1) < P
             ).astype(jnp.float32)
    infl = infl * pmask

    fnb = fnb_ref[...]
    wparts = []
    for p in range(P_PAD):
        wf = infl[:, p][:, None] * fnb
        wparts.append(jnp.sum(wf.reshape(BLK, K, IN_C), axis=1))
    weighted = jnp.concatenate(wparts, axis=1)

    conv = jnp.dot(weighted, w2_ref[...],
                   preferred_element_type=jnp.float32)

    nsum = jnp.sum(fnb, axis=1)
    valid = (jnp.abs(nsum) > 0.0).astype(jnp.float32).reshape(BLK, K)
    ncount = jnp.maximum(jnp.sum(valid, axis=1), 1.0)
    conv = conv / ncount[:, None] + bias_ref[0][None, :]

    conv_ref[...] = conv

    rows = jnp.stack([jnp.sum(conv, axis=0),
                      jnp.sum(conv * conv, axis=0)], axis=0)
    rows = jnp.concatenate(
        [rows, jnp.zeros((2, 128 - OUT_C), jnp.float32)], axis=1)
    s = jnp.concatenate([rows, jnp.zeros((6, 128), jnp.float32)], axis=0)

    @pl.when(i == 0)
    def _():
        sums_ref[...] = s

    @pl.when(i != 0)
    def _():
        sums_ref[...] += s


def _stage2_kernel(conv_ref, sums_ref, gamma_ref, beta_ref, out_ref):
    mean = sums_ref[0, :OUT_C] / float(Q)
    var = sums_ref[1, :OUT_C] / float(Q) - mean * mean
    inv = lax.rsqrt(var + EPS)
    xn = (conv_ref[...] - mean[None, :]) * inv[None, :]
    xn = xn * gamma_ref[0][None, :] + beta_ref[0][None, :]
    out_ref[...] = jnp.maximum(xn, 0.0)


def kernel(query, support, features, neighbors, kernel_points, weights,
           bias, gamma, beta):
    nb1d = (neighbors.astype(jnp.int32)
            + jnp.arange(B, dtype=jnp.int32)[:, None, None] * M
            ).reshape(ROWS)
    f_flat = features.reshape(B * M, IN_C)
    s_pad = jnp.concatenate(
        [support.reshape(B * M, 3),
         jnp.zeros((B * M, 13), jnp.float32)], axis=1)

    fnb, snb = _make_sc_gather()(f_flat, s_pad, nb1d)

    q16 = jnp.concatenate(
        [query.reshape(Q, 3), jnp.zeros((Q, 13), jnp.float32)], axis=1)

    kpt = jnp.zeros((16, 16), dtype=jnp.float32)
    kpt = kpt.at[:3, :P].set(kernel_points.T)
    kpn = jnp.zeros((8, 16), dtype=jnp.float32)
    kpn = kpn.at[0, :P].set(jnp.sum(kernel_points * kernel_points, axis=1))

    w2 = jnp.concatenate(
        [weights.reshape(P * IN_C, OUT_C),
         jnp.zeros((IN_C, OUT_C), dtype=jnp.float32)], axis=0)

    bias2 = jnp.zeros((8, OUT_C), dtype=jnp.float32).at[0].set(bias)
    gamma2 = jnp.zeros((8, OUT_C), dtype=jnp.float32).at[0].set(gamma)
    beta2 = jnp.zeros((8, OUT_C), dtype=jnp.float32).at[0].set(beta)

    conv, sums = pl.pallas_call(
        _stage1_kernel,
        grid=(NBLK,),
        in_specs=[
            pl.BlockSpec((BLK, 16), lambda i: (i, 0)),
            pl.BlockSpec((BLK * K, 16), lambda i: (i, 0)),
            pl.BlockSpec((BLK * K, IN_C), lambda i: (i, 0)),
            pl.BlockSpec((16, 16), lambda i: (0, 0)),
            pl.BlockSpec((8, 16), lambda i: (0, 0)),
            pl.BlockSpec((P_PAD * IN_C, OUT_C), lambda i: (0, 0)),
            pl.BlockSpec((8, OUT_C), lambda i: (0, 0)),
        ],
        out_specs=[
            pl.BlockSpec((BLK, OUT_C), lambda i: (i, 0)),
            pl.BlockSpec((8, 128), lambda i: (0, 0)),
        ],
        out_shape=[
            jax.ShapeDtypeStruct((Q, OUT_C), jnp.float32),
            jax.ShapeDtypeStruct((8, 128), jnp.float32),
        ],
    )(q16, snb, fnb, kpt, kpn, w2, bias2)

    out = pl.pallas_call(
        _stage2_kernel,
        grid=(NBLK,),
        in_specs=[
            pl.BlockSpec((BLK, OUT_C), lambda i: (i, 0)),
            pl.BlockSpec((8, 128), lambda i: (0, 0)),
            pl.BlockSpec((8, OUT_C), lambda i: (0, 0)),
            pl.BlockSpec((8, OUT_C), lambda i: (0, 0)),
        ],
        out_specs=pl.BlockSpec((BLK, OUT_C), lambda i: (i, 0)),
        out_shape=jax.ShapeDtypeStruct((Q, OUT_C), jnp.float32),
    )(conv, sums, gamma2, beta2)

    return out.reshape(B, N, OUT_C)


# speedup vs baseline: 13.9651x; 1.5422x over previous
import functools

import jax
import jax.numpy as jnp
from jax import lax
from jax.experimental import pallas as pl
from jax.experimental.pallas import tpu as pltpu
from jax.experimental.pallas import tpu_sc as plsc

B = 2
N = 8192
M = 8192
K = 32
P = 15
P_PAD = 16
IN_C = 64
OUT_C = 64
SIGMA = 1.0
EPS = 1e-5

Q = B * N
ROWS = Q * K
NW = 32
RPW = ROWS // NW
CH = 128
NCH = RPW // CH

BLK = 64
NBLK = Q // BLK


def _sc_gather_body(f_hbm, s_hbm, nb_hbm, fnb_hbm, snb_hbm,
                    idx_v, frows, srows, semf, sems):
    wid = lax.axis_index("s") * 2 + lax.axis_index("c")
    base0 = wid * RPW

    def body(j, carry):
        base = base0 + j * CH
        pltpu.sync_copy(nb_hbm.at[pl.ds(base, CH)], idx_v)
        cf = pltpu.async_copy(f_hbm.at[idx_v], frows, semf)
        cs = pltpu.async_copy(s_hbm.at[idx_v], srows, sems)
        cf.wait()
        cs.wait()
        pltpu.sync_copy(frows, fnb_hbm.at[pl.ds(base, CH)])
        pltpu.sync_copy(srows, snb_hbm.at[pl.ds(base, CH)])
        return carry

    lax.fori_loop(0, NCH, body, 0)


def _make_sc_gather():
    return pl.kernel(
        _sc_gather_body,
        out_type=[jax.ShapeDtypeStruct((ROWS, IN_C), jnp.float32),
                  jax.ShapeDtypeStruct((ROWS, 16), jnp.float32)],
        mesh=plsc.VectorSubcoreMesh(core_axis_name="c",
                                    subcore_axis_name="s"),
        scratch_types=[pltpu.VMEM((CH,), jnp.int32),
                       pltpu.VMEM((CH, IN_C), jnp.float32),
                       pltpu.VMEM((CH, 16), jnp.float32),
                       pltpu.SemaphoreType.DMA,
                       pltpu.SemaphoreType.DMA],
        compiler_params=pltpu.CompilerParams(use_tc_tiling_on_sc=False),
    )


def _stage1_kernel(q_ref, snb_ref, fnb_ref, kpt_ref, kpn_ref, exp_ref,
                   w2_ref, bias_ref, conv_ref, sums_ref):
    i = pl.program_id(0)

    qrep = jnp.broadcast_to(q_ref[...][:, None, :],
                            (BLK, K, 16)).reshape(BLK * K, 16)
    d3 = snb_ref[...] - qrep

    cross = jnp.dot(d3, kpt_ref[...],
                    preferred_element_type=jnp.float32)
    nrm = jnp.sum(d3 * d3, axis=1, keepdims=True)
    sq = jnp.maximum(nrm + kpn_ref[0][None, :] - 2.0 * cross, 0.0)
    infl = jnp.maximum(1.0 - jnp.sqrt(sq) / SIGMA, 0.0)

    inflx = jnp.dot(infl, exp_ref[...],
                    preferred_element_type=jnp.float32)
    fnb = fnb_ref[...]
    fnbx = jnp.concatenate([fnb] * P_PAD, axis=1)
    wf = inflx * fnbx
    weighted = jnp.sum(wf.reshape(BLK, K, P_PAD * IN_C), axis=1)

    conv = jnp.dot(weighted, w2_ref[...],
                   preferred_element_type=jnp.float32)

    nsum = jnp.sum(fnb, axis=1)
    valid = (jnp.abs(nsum) > 0.0).astype(jnp.float32).reshape(BLK, K)
    ncount = jnp.maximum(jnp.sum(valid, axis=1), 1.0)
    conv = conv / ncount[:, None] + bias_ref[0][None, :]

    conv_ref[...] = conv

    rows = jnp.stack([jnp.sum(conv, axis=0),
                      jnp.sum(conv * conv, axis=0)], axis=0)
    rows = jnp.concatenate(
        [rows, jnp.zeros((2, 128 - OUT_C), jnp.float32)], axis=1)
    s = jnp.concatenate([rows, jnp.zeros((6, 128), jnp.float32)], axis=0)

    @pl.when(i == 0)
    def _():
        sums_ref[...] = s

    @pl.when(i != 0)
    def _():
        sums_ref[...] += s


def _stage2_kernel(conv_ref, sums_ref, gamma_ref, beta_ref, out_ref):
    mean = sums_ref[0, :OUT_C] / float(Q)
    var = sums_ref[1, :OUT_C] / float(Q) - mean * mean
    inv = lax.rsqrt(var + EPS)
    xn = (conv_ref[...] - mean[None, :]) * inv[None, :]
    xn = xn * gamma_ref[0][None, :] + beta_ref[0][None, :]
    out_ref[...] = jnp.maximum(xn, 0.0)


def kernel(query, support, features, neighbors, kernel_points, weights,
           bias, gamma, beta):
    nb1d = (neighbors.astype(jnp.int32)
            + jnp.arange(B, dtype=jnp.int32)[:, None, None] * M
            ).reshape(ROWS)
    f_flat = features.reshape(B * M, IN_C)
    s_pad = jnp.concatenate(
        [support.reshape(B * M, 3),
         jnp.zeros((B * M, 13), jnp.float32)], axis=1)

    fnb, snb = _make_sc_gather()(f_flat, s_pad, nb1d)

    q16 = jnp.concatenate(
        [query.reshape(Q, 3), jnp.zeros((Q, 13), jnp.float32)], axis=1)

    kpt = jnp.zeros((16, 16), dtype=jnp.float32)
    kpt = kpt.at[:3, :P].set(kernel_points.T)
    kpn = jnp.zeros((8, 16), dtype=jnp.float32)
    kpn = kpn.at[0, :P].set(jnp.sum(kernel_points * kernel_points, axis=1))
    expand = jnp.zeros((16, P_PAD * IN_C), dtype=jnp.float32)
    for p in range(P):
        expand = expand.at[p, p * IN_C:(p + 1) * IN_C].set(1.0)

    w2 = jnp.concatenate(
        [weights.reshape(P * IN_C, OUT_C),
         jnp.zeros((IN_C, OUT_C), dtype=jnp.float32)], axis=0)

    bias2 = jnp.zeros((8, OUT_C), dtype=jnp.float32).at[0].set(bias)
    gamma2 = jnp.zeros((8, OUT_C), dtype=jnp.float32).at[0].set(gamma)
    beta2 = jnp.zeros((8, OUT_C), dtype=jnp.float32).at[0].set(beta)

    conv, sums = pl.pallas_call(
        _stage1_kernel,
        grid=(NBLK,),
        in_specs=[
            pl.BlockSpec((BLK, 16), lambda i: (i, 0)),
            pl.BlockSpec((BLK * K, 16), lambda i: (i, 0)),
            pl.BlockSpec((BLK * K, IN_C), lambda i: (i, 0)),
            pl.BlockSpec((16, 16), lambda i: (0, 0)),
            pl.BlockSpec((8, 16), lambda i: (0, 0)),
            pl.BlockSpec((16, P_PAD * IN_C), lambda i: (0, 0)),
            pl.BlockSpec((P_PAD * IN_C, OUT_C), lambda i: (0, 0)),
            pl.BlockSpec((8, OUT_C), lambda i: (0, 0)),
        ],
        out_specs=[
            pl.BlockSpec((BLK, OUT_C), lambda i: (i, 0)),
            pl.BlockSpec((8, 128), lambda i: (0, 0)),
        ],
        out_shape=[
            jax.ShapeDtypeStruct((Q, OUT_C), jnp.float32),
            jax.ShapeDtypeStruct((8, 128), jnp.float32),
        ],
    )(q16, snb, fnb, kpt, kpn, expand, w2, bias2)

    out = pl.pallas_call(
        _stage2_kernel,
        grid=(NBLK,),
        in_specs=[
            pl.BlockSpec((BLK, OUT_C), lambda i: (i, 0)),
            pl.BlockSpec((8, 128), lambda i: (0, 0)),
            pl.BlockSpec((8, OUT_C), lambda i: (0, 0)),
            pl.BlockSpec((8, OUT_C), lambda i: (0, 0)),
        ],
        out_specs=pl.BlockSpec((BLK, OUT_C), lambda i: (i, 0)),
        out_shape=jax.ShapeDtypeStruct((Q, OUT_C), jnp.float32),
    )(conv, sums, gamma2, beta2)

    return out.reshape(B, N, OUT_C)


# speedup vs baseline: 14.8178x; 1.0611x over previous
import functools

import jax
import jax.numpy as jnp
from jax import lax
from jax.experimental import pallas as pl
from jax.experimental.pallas import tpu as pltpu
from jax.experimental.pallas import tpu_sc as plsc

B = 2
N = 8192
M = 8192
K = 32
P = 15
P_PAD = 16
IN_C = 64
OUT_C = 64
SIGMA = 1.0
EPS = 1e-5

Q = B * N
ROWS = Q * K
NW = 32
RPW = ROWS // NW
CH = 128
NCH = RPW // CH

BLK = 128
NBLK = Q // BLK


def _sc_gather_body(f_hbm, s_hbm, nb_hbm, fnb_hbm, snb_hbm,
                    idx_v, frows, srows, semf, sems):
    wid = lax.axis_index("s") * 2 + lax.axis_index("c")
    base0 = wid * RPW

    def body(j, carry):
        base = base0 + j * CH
        pltpu.sync_copy(nb_hbm.at[pl.ds(base, CH)], idx_v)
        cf = pltpu.async_copy(f_hbm.at[idx_v], frows, semf)
        cs = pltpu.async_copy(s_hbm.at[idx_v], srows, sems)
        cf.wait()
        cs.wait()
        pltpu.sync_copy(frows, fnb_hbm.at[pl.ds(base, CH)])
        pltpu.sync_copy(srows, snb_hbm.at[pl.ds(base, CH)])
        return carry

    lax.fori_loop(0, NCH, body, 0)


def _make_sc_gather():
    return pl.kernel(
        _sc_gather_body,
        out_type=[jax.ShapeDtypeStruct((ROWS, IN_C), jnp.float32),
                  jax.ShapeDtypeStruct((ROWS, 16), jnp.float32)],
        mesh=plsc.VectorSubcoreMesh(core_axis_name="c",
                                    subcore_axis_name="s"),
        scratch_types=[pltpu.VMEM((CH,), jnp.int32),
                       pltpu.VMEM((CH, IN_C), jnp.float32),
                       pltpu.VMEM((CH, 16), jnp.float32),
                       pltpu.SemaphoreType.DMA,
                       pltpu.SemaphoreType.DMA],
        compiler_params=pltpu.CompilerParams(use_tc_tiling_on_sc=False),
    )


def _stage1_kernel(q_ref, snb_ref, fnb_ref, kpt_ref, kpn_ref, exp_ref,
                   w2_ref, bias_ref, conv_ref, sums_ref):
    i = pl.program_id(0)

    qrep = jnp.broadcast_to(q_ref[...][:, None, :],
                            (BLK, K, 16)).reshape(BLK * K, 16)
    d3 = snb_ref[...] - qrep

    cross = jnp.dot(d3, kpt_ref[...],
                    preferred_element_type=jnp.float32)
    nrm = jnp.sum(d3 * d3, axis=1, keepdims=True)
    sq = jnp.maximum(nrm + kpn_ref[0][None, :] - 2.0 * cross, 0.0)
    infl = jnp.maximum(1.0 - jnp.sqrt(sq) / SIGMA, 0.0)

    inflx = jnp.dot(infl, exp_ref[...],
                    preferred_element_type=jnp.float32)
    fnb = fnb_ref[...]
    fnbx = jnp.concatenate([fnb] * P_PAD, axis=1)
    wf = inflx * fnbx
    weighted = jnp.sum(wf.reshape(BLK, K, P_PAD * IN_C), axis=1)

    conv = jnp.dot(weighted, w2_ref[...],
                   preferred_element_type=jnp.float32)

    nsum = jnp.sum(fnb, axis=1)
    valid = (jnp.abs(nsum) > 0.0).astype(jnp.float32).reshape(BLK, K)
    ncount = jnp.maximum(jnp.sum(valid, axis=1), 1.0)
    conv = conv / ncount[:, None] + bias_ref[0][None, :]

    conv_ref[...] = conv

    rows = jnp.stack([jnp.sum(conv, axis=0),
                      jnp.sum(conv * conv, axis=0)], axis=0)
    rows = jnp.concatenate(
        [rows, jnp.zeros((2, 128 - OUT_C), jnp.float32)], axis=1)
    s = jnp.concatenate([rows, jnp.zeros((6, 128), jnp.float32)], axis=0)

    @pl.when(i == 0)
    def _():
        sums_ref[...] = s

    @pl.when(i != 0)
    def _():
        sums_ref[...] += s


def _stage2_kernel(conv_ref, sums_ref, gamma_ref, beta_ref, out_ref):
    mean = sums_ref[0, :OUT_C] / float(Q)
    var = sums_ref[1, :OUT_C] / float(Q) - mean * mean
    inv = lax.rsqrt(var + EPS)
    xn = (conv_ref[...] - mean[None, :]) * inv[None, :]
    xn = xn * gamma_ref[0][None, :] + beta_ref[0][None, :]
    out_ref[...] = jnp.maximum(xn, 0.0)


def kernel(query, support, features, neighbors, kernel_points, weights,
           bias, gamma, beta):
    nb1d = (neighbors.astype(jnp.int32)
            + jnp.arange(B, dtype=jnp.int32)[:, None, None] * M
            ).reshape(ROWS)
    f_flat = features.reshape(B * M, IN_C)
    s_pad = jnp.concatenate(
        [support.reshape(B * M, 3),
         jnp.zeros((B * M, 13), jnp.float32)], axis=1)

    fnb, snb = _make_sc_gather()(f_flat, s_pad, nb1d)

    q16 = jnp.concatenate(
        [query.reshape(Q, 3), jnp.zeros((Q, 13), jnp.float32)], axis=1)

    kpt = jnp.zeros((16, 16), dtype=jnp.float32)
    kpt = kpt.at[:3, :P].set(kernel_points.T)
    kpn = jnp.zeros((8, 16), dtype=jnp.float32)
    kpn = kpn.at[0, :P].set(jnp.sum(kernel_points * kernel_points, axis=1))
    expand = jnp.zeros((16, P_PAD * IN_C), dtype=jnp.float32)
    for p in range(P):
        expand = expand.at[p, p * IN_C:(p + 1) * IN_C].set(1.0)

    w2 = jnp.concatenate(
        [weights.reshape(P * IN_C, OUT_C),
         jnp.zeros((IN_C, OUT_C), dtype=jnp.float32)], axis=0)

    bias2 = jnp.zeros((8, OUT_C), dtype=jnp.float32).at[0].set(bias)
    gamma2 = jnp.zeros((8, OUT_C), dtype=jnp.float32).at[0].set(gamma)
    beta2 = jnp.zeros((8, OUT_C), dtype=jnp.float32).at[0].set(beta)

    conv, sums = pl.pallas_call(
        _stage1_kernel,
        grid=(NBLK,),
        compiler_params=pltpu.CompilerParams(
            vmem_limit_bytes=110 * 1024 * 1024),
        in_specs=[
            pl.BlockSpec((BLK, 16), lambda i: (i, 0)),
            pl.BlockSpec((BLK * K, 16), lambda i: (i, 0)),
            pl.BlockSpec((BLK * K, IN_C), lambda i: (i, 0)),
            pl.BlockSpec((16, 16), lambda i: (0, 0)),
            pl.BlockSpec((8, 16), lambda i: (0, 0)),
            pl.BlockSpec((16, P_PAD * IN_C), lambda i: (0, 0)),
            pl.BlockSpec((P_PAD * IN_C, OUT_C), lambda i: (0, 0)),
            pl.BlockSpec((8, OUT_C), lambda i: (0, 0)),
        ],
        out_specs=[
            pl.BlockSpec((BLK, OUT_C), lambda i: (i, 0)),
            pl.BlockSpec((8, 128), lambda i: (0, 0)),
        ],
        out_shape=[
            jax.ShapeDtypeStruct((Q, OUT_C), jnp.float32),
            jax.ShapeDtypeStruct((8, 128), jnp.float32),
        ],
    )(q16, snb, fnb, kpt, kpn, expand, w2, bias2)

    out = pl.pallas_call(
        _stage2_kernel,
        grid=(NBLK,),
        in_specs=[
            pl.BlockSpec((BLK, OUT_C), lambda i: (i, 0)),
            pl.BlockSpec((8, 128), lambda i: (0, 0)),
            pl.BlockSpec((8, OUT_C), lambda i: (0, 0)),
            pl.BlockSpec((8, OUT_C), lambda i: (0, 0)),
        ],
        out_specs=pl.BlockSpec((BLK, OUT_C), lambda i: (i, 0)),
        out_shape=jax.ShapeDtypeStruct((Q, OUT_C), jnp.float32),
    )(conv, sums, gamma2, beta2)

    return out.reshape(B, N, OUT_C)
